# baseline (speedup 1.0000x reference)
import functools

import jax
import jax.numpy as jnp
from jax import lax
from jax.experimental import pallas as pl
from jax.experimental.pallas import tpu as pltpu
from jax.experimental.pallas import tpu_sc as plsc

VOCAB = 1000000
DIM = 32
NUM_VALUES = 327680
BATCH = 16384

NC = 2
NS = 16
NW = NC * NS
B_PER_W = NUM_VALUES // NW
K = 10
CHUNK = K * 128
NCH = B_PER_W // CHUNK
LPW = BATCH // NW
OFFW = LPW + 8

_mesh = plsc.VectorSubcoreMesh(core_axis_name="c", subcore_axis_name="s")


@functools.partial(
    pl.kernel,
    mesh=_mesh,
    compiler_params=pltpu.CompilerParams(use_tc_tiling_on_sc=False),
    out_type=(
        jax.ShapeDtypeStruct((NW, NCH, K, 128, DIM), jnp.float32),
        jax.ShapeDtypeStruct((NW, LPW), jnp.int32),
    ),
    scratch_types=[
        pltpu.VMEM((2, K, 128), jnp.int32),
        pltpu.VMEM((2, K, 128, DIM), jnp.float32),
        pltpu.VMEM((OFFW,), jnp.int32),
        pltpu.VMEM((LPW,), jnp.int32),
        pltpu.SemaphoreType.DMA,
        pltpu.SemaphoreType.DMA,
        pltpu.SemaphoreType.DMA,
        pltpu.SemaphoreType.DMA,
    ],
)
def _sc_lookup(values_h, offsets_h, table_h, emb_h, len_h,
               idx_v, rows_v, off_v, len_v, gsem0, gsem1, osem0, osem1):
    wid = lax.axis_index("s") * NC + lax.axis_index("c")
    gsem = (gsem0, gsem1)
    osem = (osem0, osem1)

    def fire(g, b):
        pltpu.sync_copy(values_h.at[wid, g], idx_v.at[b])
        return [
            pltpu.async_copy(table_h.at[idx_v.at[b, j]], rows_v.at[b, j], gsem[b])
            for j in range(K)
        ]

    gh = [None, None]
    oh = [None, None]
    gh[0] = fire(0, 0)

    pltpu.sync_copy(offsets_h.at[pl.ds(wid * LPW, OFFW)], off_v)
    for j in range(LPW // 16):
        a = off_v[pl.ds(j * 16, 16)]
        b = off_v[pl.ds(j * 16 + 1, 16)]
        len_v[pl.ds(j * 16, 16)] = b - a
    pltpu.sync_copy(len_v, len_h.at[wid])

    for g in range(NCH):
        cur = g & 1
        nxt = cur ^ 1
        if g + 1 < NCH:
            if oh[nxt] is not None:
                oh[nxt].wait()
            gh[nxt] = fire(g + 1, nxt)
        for h in gh[cur]:
            h.wait()
        oh[cur] = pltpu.async_copy(rows_v.at[cur], emb_h.at[wid, g], osem[cur])
    oh[0].wait()
    oh[1].wait()


def kernel(values, offsets, table):
    values_r = values.astype(jnp.int32).reshape(NW, NCH, K, 128)
    offsets_p = jnp.pad(offsets.astype(jnp.int32), (0, NW * LPW + OFFW - (BATCH + 1)))
    emb, lens = _sc_lookup(values_r, offsets_p, table)
    return emb.reshape(NUM_VALUES, DIM), lens.reshape(BATCH)
